# baseline (speedup 1.0000x reference)
import functools

import jax
import jax.numpy as jnp
import numpy as np
from jax import lax
from jax.experimental import pallas as pl
from jax.experimental.pallas import tpu as pltpu
from jax.experimental.pallas import tpu_sc as plsc

_INTERPRET = False

_NP = 64
_H = 8
_DH = 16
_BB = 8
_NW = 32
_CHUNK = 128


def _sc_gather(table, idx):
    nw, nch, chunk = idx.shape
    tot = nw * nch * chunk
    d = table.shape[1]
    perw = nch * chunk
    mesh = plsc.VectorSubcoreMesh(core_axis_name="c", subcore_axis_name="s")

    @functools.partial(
        pl.kernel,
        mesh=mesh,
        out_type=jax.ShapeDtypeStruct((tot, d), jnp.float32),
        scratch_types=[
            pltpu.VMEM((nch, chunk), jnp.int32),
            pltpu.VMEM((chunk, d), jnp.float32),
            pltpu.SemaphoreType.DMA,
        ],
    )
    def k(table_hbm, idx_hbm, out_hbm, idx_v, rows_v, sem):
        wid = lax.axis_index("s") * 2 + lax.axis_index("c")
        base = wid * perw
        pltpu.sync_copy(idx_hbm.at[wid], idx_v)

        def body(j, carry):
            pltpu.async_copy(table_hbm.at[idx_v.at[j]], rows_v, sem).wait()
            pltpu.sync_copy(rows_v, out_hbm.at[pl.ds(base + j * chunk, chunk)])
            return carry

        lax.fori_loop(0, nch, body, 0)

    return k(table, idx)


def _ln(x):
    m = jnp.mean(x, axis=-1, keepdims=True)
    v = jnp.mean((x - m) ** 2, axis=-1, keepdims=True)
    return (x - m) * lax.rsqrt(v + 1e-5)


def _chan_body(hg_ref, adj_ref, aliasT_ref, la_ref, pos_ref,
               wq_ref, wk_ref, wv_ref, wo_ref, wf1_ref, wf2_ref, out_ref):
    f32 = jnp.float32
    NPP = _NP
    BB = _BB
    D = hg_ref.shape[-1]
    depth = wq_ref.shape[1]
    pos = pos_ref[...]
    la = la_ref[0]

    iota_r = lax.broadcasted_iota(jnp.int32, (NPP, NPP), 0)
    iota_c = lax.broadcasted_iota(jnp.int32, (NPP, NPP), 1)

    hs_list = []
    for b in range(BB):
        hb = hg_ref[0, b]
        ab = adj_ref[b]
        ha = jnp.concatenate([hb * la[r:r + 1, :] for r in range(4)], axis=0)
        sc4 = lax.dot_general(ha, hb, (((1,), (1,)), ((), ())),
                              preferred_element_type=f32)
        rel = jnp.clip(ab - 1, 0, 3)
        e = sc4[0:NPP]
        for r in range(1, 4):
            e = jnp.where(rel == r, sc4[r * NPP:(r + 1) * NPP], e)
        e = jnp.where(e >= 0, e, 0.2 * e)
        e = jnp.where(ab > 0, e, -1e9)
        mx = jnp.max(e, axis=-1, keepdims=True)
        p = jnp.exp(e - mx)
        attn = p / jnp.sum(p, axis=-1, keepdims=True)
        hasrow = jnp.any(ab > 0, axis=-1, keepdims=True)
        attn = jnp.where(hasrow, attn, 0.0)
        hagg = lax.dot_general(attn, hb, (((1,), (0,)), ((), ())),
                               preferred_element_type=f32)
        alc = aliasT_ref[0][:, b:b + 1]
        onehot = (alc == iota_c).astype(f32)
        hs_b = lax.dot_general(onehot, hagg, (((1,), (0,)), ((), ())),
                               preferred_element_type=f32)
        hs_list.append(_ln(hs_b + pos))
    x = jnp.concatenate(hs_list, axis=0)

    hm_r = lax.broadcasted_iota(jnp.int32, (_H * NPP, D), 0) // NPP
    hm_c = lax.broadcasted_iota(jnp.int32, (_H * NPP, D), 1) // _DH
    headmask = (hm_r == hm_c).astype(f32)
    lg_r = lax.broadcasted_iota(jnp.int32, (NPP, _H * NPP), 0)
    lg_c = lax.broadcasted_iota(jnp.int32, (NPP, _H * NPP), 1)
    lg_m = lg_c % NPP
    attn_ok = (lg_m <= lg_r) & (lg_m < 50)
    bm_r = lax.broadcasted_iota(jnp.int32, (_H * NPP, _H), 0) // NPP
    bm_c = lax.broadcasted_iota(jnp.int32, (_H * NPP, _H), 1)
    bm8 = (bm_r == bm_c).astype(f32)
    scale = 1.0 / np.sqrt(_DH)

    for l in range(depth):
        xl = _ln(x)
        q = jnp.dot(xl, wq_ref[0, l], preferred_element_type=f32)
        kk = jnp.dot(xl, wk_ref[0, l], preferred_element_type=f32)
        v = jnp.dot(xl, wv_ref[0, l], preferred_element_type=f32)
        o_list = []
        for b in range(BB):
            qb = q[b * NPP:(b + 1) * NPP]
            kb = kk[b * NPP:(b + 1) * NPP]
            vb = v[b * NPP:(b + 1) * NPP]
            khat = jnp.concatenate([kb] * _H, axis=0) * headmask
            vhat = jnp.concatenate([vb] * _H, axis=0) * headmask
            logits = lax.dot_general(qb, khat, (((1,), (1,)), ((), ())),
                                     preferred_element_type=f32) * scale
            logits = jnp.where(attn_ok, logits, -1e9)
            mxl = jnp.max(logits, axis=-1, keepdims=True)
            pl_ = jnp.exp(logits - mxl)
            ssum = jnp.dot(pl_, bm8, preferred_element_type=f32)
            sfull = lax.dot_general(ssum, bm8, (((1,), (1,)), ((), ())),
                                    preferred_element_type=f32)
            aw = pl_ / sfull
            o_list.append(jnp.dot(aw, vhat, preferred_element_type=f32))
        o = jnp.concatenate(o_list, axis=0)
        x = x + jnp.dot(o, wo_ref[0, l], preferred_element_type=f32)
        xf = _ln(x)
        hff = jax.nn.gelu(jnp.dot(xf, wf1_ref[0, l], preferred_element_type=f32))
        x = x + jnp.dot(hff, wf2_ref[0, l], preferred_element_type=f32)

    p_r = lax.broadcasted_iota(jnp.int32, (BB, BB * NPP), 0)
    p_c = lax.broadcasted_iota(jnp.int32, (BB, BB * NPP), 1)
    pool = jnp.where((p_c // NPP == p_r) & (p_c % NPP < 50), 1.0 / 50.0, 0.0)
    out_ref[0] = jnp.dot(pool, x, preferred_element_type=f32)


def _run_channels(hg, adj_p, aliasT, la_a, pos_p, Wq, Wk, Wv, Wo, Wff1, Wff2):
    NK, B = hg.shape[0], hg.shape[1]
    D = hg.shape[-1]
    depth = Wq.shape[1]
    grid = (NK, B // _BB)
    return pl.pallas_call(
        _chan_body,
        grid=grid,
        in_specs=[
            pl.BlockSpec((1, _BB, _NP, D), lambda k, g: (k, g, 0, 0)),
            pl.BlockSpec((_BB, _NP, _NP), lambda k, g: (g, 0, 0)),
            pl.BlockSpec((1, _NP, _BB), lambda k, g: (g, 0, 0)),
            pl.BlockSpec((1, 4, D), lambda k, g: (k, 0, 0)),
            pl.BlockSpec((_NP, D), lambda k, g: (0, 0)),
            pl.BlockSpec((1, depth, D, D), lambda k, g: (k, 0, 0, 0)),
            pl.BlockSpec((1, depth, D, D), lambda k, g: (k, 0, 0, 0)),
            pl.BlockSpec((1, depth, D, D), lambda k, g: (k, 0, 0, 0)),
            pl.BlockSpec((1, depth, D, D), lambda k, g: (k, 0, 0, 0)),
            pl.BlockSpec((1, depth, D, 4 * D), lambda k, g: (k, 0, 0, 0)),
            pl.BlockSpec((1, depth, 4 * D, D), lambda k, g: (k, 0, 0, 0)),
        ],
        out_specs=pl.BlockSpec((1, _BB, D), lambda k, g: (k, g, 0)),
        out_shape=jax.ShapeDtypeStruct((NK, B, D), jnp.float32),
        compiler_params=pltpu.CompilerParams(
            dimension_semantics=("arbitrary", "arbitrary")),
        interpret=_INTERPRET,
    )(hg, adj_p, aliasT, la_a, pos_p, Wq, Wk, Wv, Wo, Wff1, Wff2)


def _gate_body(reps_ref, wg_ref, wn_ref, eps_ref, out_ref):
    f32 = jnp.float32
    NK = reps_ref.shape[0]
    B, D = reps_ref.shape[1], reps_ref.shape[2]
    xg = reps_ref[0]
    lg = lax.dot_general(xg, wg_ref[...], (((1,), (1,)), ((), ())),
                         preferred_element_type=f32)
    sp_in = lax.dot_general(xg, wn_ref[...], (((1,), (1,)), ((), ())),
                            preferred_element_type=f32)
    sp = jnp.maximum(sp_in, 0.0) + jnp.log1p(jnp.exp(-jnp.abs(sp_in)))
    noisy = lg + eps_ref[...] * sp
    lane = lax.broadcasted_iota(jnp.int32, (B, D), 1)
    noisy = jnp.where(lane < NK, noisy, -1e30)
    m1 = jnp.max(noisy, axis=-1, keepdims=True)
    i1 = jnp.min(jnp.where(noisy == m1, lane, 999), axis=-1, keepdims=True)
    n2 = jnp.where(lane == i1, -1e30, noisy)
    m2 = jnp.max(n2, axis=-1, keepdims=True)
    i2 = jnp.min(jnp.where(n2 == m2, lane, 999), axis=-1, keepdims=True)
    e2 = jnp.exp(m2 - m1)
    w1 = 1.0 / (1.0 + e2)
    w2 = e2 / (1.0 + e2)
    sess = jnp.zeros((B, D), f32)
    for k in range(NK):
        gk = jnp.sum(jnp.where(lane == k,
                               jnp.where(i1 == k, w1,
                                         jnp.where(i2 == k, w2, 0.0)),
                               0.0), axis=-1, keepdims=True)
        sess = sess + gk * reps_ref[k]
    out_ref[...] = _ln(sess)


def _run_gate(reps, wg_p, wn_p, eps_p):
    B, D = reps.shape[1], reps.shape[2]
    return pl.pallas_call(
        _gate_body,
        out_shape=jax.ShapeDtypeStruct((B, D), jnp.float32),
        interpret=_INTERPRET,
    )(reps, wg_p, wn_p, eps_p)


def _logits_body(sess_ref, emb_ref, out_ref):
    out_ref[...] = lax.dot_general(sess_ref[...], emb_ref[...],
                                   (((1,), (1,)), ((), ())),
                                   preferred_element_type=jnp.float32)


def _run_logits(sess, emb_item, vblk=2048):
    B, D = sess.shape
    V = emb_item.shape[0]
    grid = (pl.cdiv(V, vblk),)
    return pl.pallas_call(
        _logits_body,
        grid=grid,
        in_specs=[
            pl.BlockSpec((B, D), lambda i: (0, 0)),
            pl.BlockSpec((vblk, D), lambda i: (i, 0)),
        ],
        out_specs=pl.BlockSpec((B, vblk), lambda i: (0, i)),
        out_shape=jax.ShapeDtypeStruct((B, V), jnp.float32),
        compiler_params=pltpu.CompilerParams(
            dimension_semantics=("arbitrary",)),
        interpret=_INTERPRET,
    )(sess, emb_item)


def kernel(items_seq, items_attr, adj, alias, mask, emb_item, emb_attr,
           pos_emb, la_a, Wq, Wk, Wv, Wo, Wff1, Wff2, Wg, Wn):
    B, N = items_seq.shape
    L = alias.shape[1]
    NK = la_a.shape[0]
    D = emb_item.shape[1]
    VI = emb_item.shape[0]
    VA = emb_attr.shape[1]

    table = jnp.concatenate([emb_item, emb_attr.reshape(-1, D)], axis=0)
    pad_n = _NP - N
    idx0 = jnp.pad(items_seq, ((0, 0), (0, pad_n)))
    idxa = jnp.pad(items_attr, ((0, 0), (0, 0), (0, pad_n)))
    offs = (VI + VA * jnp.arange(NK - 1, dtype=idxa.dtype))[:, None, None]
    idx_all = jnp.concatenate([idx0[None], idxa + offs], axis=0)
    idx_all = idx_all.astype(jnp.int32).reshape(_NW, -1, _CHUNK)

    hg = _sc_gather(table, idx_all).reshape(NK, B, _NP, D)

    adj_p = jnp.pad(adj, ((0, 0), (0, pad_n), (0, pad_n))).astype(jnp.int32)
    aliasT = (jnp.pad(alias, ((0, 0), (0, _NP - L))).astype(jnp.int32)
              .reshape(B // _BB, _BB, _NP).transpose(0, 2, 1))
    pos_p = jnp.pad(pos_emb[:L], ((0, _NP - L), (0, 0)))
    reps = _run_channels(hg, adj_p, aliasT, la_a, pos_p,
                         Wq, Wk, Wv, Wo, Wff1, Wff2)

    eps = jax.random.normal(jax.random.key(42), (B, NK))
    eps_p = jnp.pad(eps, ((0, 0), (0, D - NK)))
    wg_p = jnp.pad(Wg, ((0, D - NK), (0, 0)))
    wn_p = jnp.pad(Wn, ((0, D - NK), (0, 0)))
    sess = _run_gate(reps, wg_p, wn_p, eps_p)

    return _run_logits(sess, emb_item)
